# Initial kernel scaffold; baseline (speedup 1.0000x reference)
import jax
import jax.numpy as jnp
from jax.experimental import pallas as pl


def kernel(x):
    raise NotImplementedError("write your pallas kernel here")


# speedup vs baseline: 15.6770x; 15.6770x over previous
import functools

import jax
import jax.numpy as jnp
from jax.experimental import pallas as pl
from jax.experimental.pallas import tpu as pltpu

_B, _N = 64, 32768
_K = int(0.1 * _N)
_POS = _N - _K
_ROWS_PER_BLK = 8


def _kwinners_block(x_ref, o_ref):
    _INT_MIN = jnp.int32(-2147483648)
    x = x_ref[...]
    i = jax.lax.bitcast_convert_type(x, jnp.int32)
    s = i ^ (jax.lax.shift_right_arithmetic(i, 31) & jnp.int32(0x7FFFFFFF))

    rows = x.shape[0]
    prefix_u = jnp.zeros((rows, 1), jnp.int32)
    for b in range(31, -1, -1):
        if b == 31:
            mid_u = jnp.full((rows, 1), jnp.int32(0x7FFFFFFF))
        else:
            mid_u = prefix_u | jnp.int32((1 << b) - 1)
        mid_s = mid_u ^ _INT_MIN
        c = jnp.sum((s <= mid_s).astype(jnp.int32), axis=1, keepdims=True)
        take1 = c < _POS
        bit = _INT_MIN if b == 31 else jnp.int32(1 << b)
        prefix_u = jnp.where(take1, prefix_u | bit, prefix_u)

    s_thr = prefix_u ^ _INT_MIN
    i_thr = s_thr ^ (jax.lax.shift_right_arithmetic(s_thr, 31) & jnp.int32(0x7FFFFFFF))
    thr = jax.lax.bitcast_convert_type(i_thr, jnp.float32)
    o_ref[...] = jnp.where(x > thr, x, jnp.float32(0.0))


@jax.jit
def kernel(x):
    grid = _B // _ROWS_PER_BLK
    return pl.pallas_call(
        _kwinners_block,
        grid=(grid,),
        in_specs=[pl.BlockSpec((_ROWS_PER_BLK, _N), lambda g: (g, 0))],
        out_specs=pl.BlockSpec((_ROWS_PER_BLK, _N), lambda g: (g, 0)),
        out_shape=jax.ShapeDtypeStruct((_B, _N), jnp.float32),
    )(x)
